# Initial kernel scaffold; baseline (speedup 1.0000x reference)
import jax
import jax.numpy as jnp
from jax.experimental import pallas as pl


def kernel(x, edge_index, edge_weight, W1, b1, W2, b2, Wl, bl):
    raise NotImplementedError("write your pallas kernel here")


# speedup vs baseline: 12.2230x; 12.2230x over previous
import functools

import jax
import jax.numpy as jnp
from jax import lax
from jax.experimental import pallas as pl
from jax.experimental.pallas import tpu as pltpu
from jax.experimental.pallas import tpu_sc as plsc

NN = 10000
NE = 320000
D = 128
NC = 2
NS = 16
ER = NE // D
ERH = ER // NC
ROWS_MAX = (ERH + NS - 1) // NS
NPAD = 10240
NZT = 624

_mesh = plsc.VectorSubcoreMesh(
    core_axis_name="c", subcore_axis_name="s", num_cores=NC, num_subcores=NS)


def _nrows(sid):
    return jnp.where(sid < ERH % NS, ROWS_MAX, ERH // NS)


def _deg_body(dst_hbm, ew_hbm, out_hbm, dstv, ewv, zv, acc, sem):
    del sem
    cid = lax.axis_index("c")
    sid = lax.axis_index("s")

    def zero(i, _):
        zv[pl.ds(i * 16, 16)] = jnp.zeros((16,), jnp.float32)
        return ()
    lax.fori_loop(0, 40, zero, ())
    pltpu.sync_copy(zv, acc.at[pl.ds(sid * 640, 640)])
    plsc.subcore_barrier()

    n = _nrows(sid)

    def body(i, _):
        @pl.when(i < n)
        def _():
            r = cid * ERH + i * NS + sid
            pltpu.sync_copy(dst_hbm.at[r], dstv)
            pltpu.sync_copy(ew_hbm.at[r], ewv)
            pltpu.sync_copy(ewv, acc.at[dstv], add=True)
        return ()
    lax.fori_loop(0, ROWS_MAX, body, ())

    plsc.subcore_barrier()
    pltpu.sync_copy(acc.at[pl.ds(sid * 640, 640)],
                    out_hbm.at[cid, pl.ds(sid * 640, 640)])


_deg_call = pl.kernel(
    _deg_body,
    out_type=jax.ShapeDtypeStruct((NC, NPAD), jnp.float32),
    mesh=_mesh,
    scratch_types=[
        pltpu.VMEM((D,), jnp.int32),
        pltpu.VMEM((D,), jnp.float32),
        pltpu.VMEM((640,), jnp.float32),
        pltpu.VMEM_SHARED((NPAD,), jnp.float32),
        pltpu.SemaphoreType.DMA,
    ],
)


def _prop_body(g_hbm, src_hbm, dst_hbm, ew_hbm, out_hbm,
               srcv, dstv, ewv, rows, acc, sem):
    cid = lax.axis_index("c")
    sid = lax.axis_index("s")

    def zero(i, _):
        for t in range(8):
            rows[i, pl.ds(t * 16, 16)] = jnp.zeros((16,), jnp.float32)
        return ()
    lax.fori_loop(0, D, zero, ())
    for c in range(4):
        pltpu.sync_copy(rows.at[pl.ds(0, 128)],
                        acc.at[pl.ds(sid * NZT + c * 128, 128)])
    pltpu.sync_copy(rows.at[pl.ds(0, 112)],
                    acc.at[pl.ds(sid * NZT + 512, 112)])

    @pl.when(sid == 0)
    def _():
        pltpu.sync_copy(rows.at[pl.ds(0, 16)], acc.at[pl.ds(NS * NZT, 16)])
    plsc.subcore_barrier()

    n = _nrows(sid)

    def body(i, _):
        @pl.when(i < n)
        def _():
            r = cid * ERH + i * NS + sid
            pltpu.sync_copy(src_hbm.at[r], srcv)
            pltpu.sync_copy(dst_hbm.at[r], dstv)
            pltpu.sync_copy(ew_hbm.at[r], ewv)
            pltpu.async_copy(g_hbm.at[srcv], rows, sem).wait()

            def mul(g, _):
                wv = ewv[pl.ds(g * 16, 16)]
                for j in range(16):
                    w = wv[j]
                    k = g * 16 + j
                    for t in range(8):
                        rows[k, pl.ds(t * 16, 16)] = (
                            rows[k, pl.ds(t * 16, 16)] * w)
                return ()
            lax.fori_loop(0, 8, mul, ())
            pltpu.sync_copy(rows, acc.at[dstv], add=True)
        return ()
    lax.fori_loop(0, ROWS_MAX, body, ())

    plsc.subcore_barrier()
    pltpu.sync_copy(acc.at[pl.ds(sid * NZT, NZT)],
                    out_hbm.at[cid, pl.ds(sid * NZT, NZT)])

    @pl.when(sid == 0)
    def _():
        pltpu.sync_copy(acc.at[pl.ds(NS * NZT, 16)],
                        out_hbm.at[cid, pl.ds(NS * NZT, 16)])


_prop_call = pl.kernel(
    _prop_body,
    out_type=jax.ShapeDtypeStruct((NC, NN, D), jnp.float32),
    mesh=_mesh,
    scratch_types=[
        pltpu.VMEM((D,), jnp.int32),
        pltpu.VMEM((D,), jnp.int32),
        pltpu.VMEM((D,), jnp.float32),
        pltpu.VMEM((D, D), jnp.float32),
        pltpu.VMEM_SHARED((NN, D), jnp.float32),
        pltpu.SemaphoreType.DMA,
    ],
)


_NB = 1000
_GRID = NN // _NB


def _tc1_body(degp_ref, x_ref, w1_ref, dinv_ref, g1_ref):
    deg = 1.0 + degp_ref[:, 0] + degp_ref[:, 1]
    r = lax.rsqrt(jnp.maximum(deg, 1e-12))
    r = jnp.where(deg > 0, r, 0.0)[:, None]
    dinv_ref[...] = r
    h = jnp.dot(x_ref[...], w1_ref[...], preferred_element_type=jnp.float32)
    g1_ref[...] = h * r


def _tc2_body(acc_ref, g1_ref, dinv_ref, b1_ref, w2_ref, g2_ref):
    r = dinv_ref[...]
    h = (acc_ref[0] + acc_ref[1] + g1_ref[...]) * r + b1_ref[...][None, :]
    z = jnp.maximum(h, 0.0)
    g2_ref[...] = jnp.dot(z, w2_ref[...],
                          preferred_element_type=jnp.float32) * r


def _tc3_body(acc_ref, g2_ref, dinv_ref, b2_ref, wl_ref, bl_ref, out_ref):
    h = (acc_ref[0] + acc_ref[1] + g2_ref[...]) * dinv_ref[...] \
        + b2_ref[...][None, :]
    out_ref[...] = jnp.dot(h, wl_ref[...],
                           preferred_element_type=jnp.float32) \
        + bl_ref[...][None, :]


_node_spec = pl.BlockSpec((_NB, D), lambda i: (i, 0))
_dinv_spec = pl.BlockSpec((_NB, 1), lambda i: (i, 0))
_w_spec = pl.BlockSpec((D, D), lambda i: (0, 0))
_b_spec = pl.BlockSpec((D,), lambda i: (0,))
_acc_spec = pl.BlockSpec((NC, _NB, D), lambda i: (0, i, 0))

_tc1_call = pl.pallas_call(
    _tc1_body,
    grid=(_GRID,),
    in_specs=[pl.BlockSpec((_NB, NC), lambda i: (i, 0)), _node_spec, _w_spec],
    out_specs=[_dinv_spec, _node_spec],
    out_shape=[jax.ShapeDtypeStruct((NN, 1), jnp.float32),
               jax.ShapeDtypeStruct((NN, D), jnp.float32)],
)

_tc2_call = pl.pallas_call(
    _tc2_body,
    grid=(_GRID,),
    in_specs=[_acc_spec, _node_spec, _dinv_spec, _b_spec, _w_spec],
    out_specs=_node_spec,
    out_shape=jax.ShapeDtypeStruct((NN, D), jnp.float32),
)

_tc3_call = pl.pallas_call(
    _tc3_body,
    grid=(_GRID,),
    in_specs=[_acc_spec, _node_spec, _dinv_spec, _b_spec, _w_spec, _b_spec],
    out_specs=_node_spec,
    out_shape=jax.ShapeDtypeStruct((NN, D), jnp.float32),
)


def kernel(x, edge_index, edge_weight, W1, b1, W2, b2, Wl, bl):
    ei = edge_index.astype(jnp.int32)
    src = ei[0].reshape(ER, D)
    dst = ei[1].reshape(ER, D)
    ew = edge_weight.astype(jnp.float32).reshape(ER, D)

    degp = _deg_call(dst, ew)[:, :NN].T
    dinv, g1 = _tc1_call(degp, x, W1)
    acc1 = _prop_call(g1, src, dst, ew)
    g2 = _tc2_call(acc1, g1, dinv, b1, W2)
    acc2 = _prop_call(g2, src, dst, ew)
    return _tc3_call(acc2, g2, dinv, b2, Wl, bl)
